# Initial kernel scaffold; baseline (speedup 1.0000x reference)
import jax
import jax.numpy as jnp
from jax.experimental import pallas as pl


def kernel(gene, cna, mutation, feature_drug, cell_drug_adj, Wg, bg, Wc, bc, Wm, bm, Wd, bd, att_W1, att_b1, att_W2, att_b2, static_w, cW1, cb1, cW2, cb2, dW1, db1, dW2, db2, k2c_W, k2c_b, k2d_W, k2d_b, enc_W1, enc_W2, dec_W):
    raise NotImplementedError("write your pallas kernel here")


# speedup vs baseline: 7.7263x; 7.7263x over previous
import jax
import jax.numpy as jnp
from jax.experimental import pallas as pl
from jax.experimental.pallas import tpu as pltpu

N_CELL = 2048
N_DRUG = 1024
TILE = 256
TOPK = 10
F32 = jnp.float32


def _dot(a, b):
    return jax.lax.dot_general(a, b, (((1,), (0,)), ((), ())),
                               preferred_element_type=F32)


def _dot_t(a, b):
    return jax.lax.dot_general(a, b, (((1,), (1,)), ((), ())),
                               preferred_element_type=F32)


def _prep_kernel(gene, cna, mut, fd,
                 Wg, bg, Wc, bc, Wm, bm, Wd, bd,
                 aW1, ab1, aW2, ab2, sw,
                 cW1, cb1, cW2, cb2, dW1, db1, dW2, db2,
                 gene_n, cna_n, coef, catt, datt, gsq, mrs, drs):
    g = gene[...]
    gm = jnp.mean(g, axis=0, keepdims=True)
    gv = jnp.mean((g - gm) * (g - gm), axis=0, keepdims=True)
    gn = (g - gm) / (jnp.sqrt(gv) + 1e-8)
    gene_n[...] = gn
    c = cna[...]
    cm = jnp.mean(c, axis=0, keepdims=True)
    cv = jnp.mean((c - cm) * (c - cm), axis=0, keepdims=True)
    cn = (c - cm) / (jnp.sqrt(cv) + 1e-8)
    cna_n[...] = cn
    m = mut[...]
    ge = _dot(gn, Wg[...]) + bg[...]
    ce = _dot(cn, Wc[...]) + bc[...]
    me = _dot(m, Wm[...]) + bm[...]
    shared = (ge + ce + me) / 3.0
    logits = _dot(jnp.maximum(_dot(shared, aW1[...]) + ab1[...], 0.0),
                  aW2[...]) + ab2[...]
    z = logits - jnp.max(logits, axis=1, keepdims=True)
    e = jnp.exp(z)
    dyn = e / jnp.sum(e, axis=1, keepdims=True)
    swv = sw[...]
    se = jnp.exp(swv - jnp.max(swv, axis=1, keepdims=True))
    st = se / jnp.sum(se, axis=1, keepdims=True)
    coef[...] = dyn * st
    s = jax.nn.sigmoid(_dot(jnp.maximum(_dot(shared, cW1[...]) + cb1[...], 0.0),
                            cW2[...]) + cb2[...])
    catt[...] = s / (jnp.sum(s) + 1e-8)
    f = fd[...]
    de = _dot(f, Wd[...]) + bd[...]
    s2 = jax.nn.sigmoid(_dot(jnp.maximum(_dot(de, dW1[...]) + db1[...], 0.0),
                             dW2[...]) + db2[...])
    datt[...] = s2 / (jnp.sum(s2) + 1e-8)
    gsq[...] = jnp.sum(gn * gn, axis=1, keepdims=True)
    mrs[...] = jnp.sum(m, axis=1, keepdims=True)
    drs[...] = jnp.sum(f, axis=1, keepdims=True)


def _cell_kernels(gt, gf, ct, cf, mt, mf, gsq_t, gsq_r, mrs_t, mrs_r):
    d2 = jnp.maximum(gsq_t[...] + gsq_r[...] - 2.0 * _dot_t(gt[...], gf[...]),
                     0.0)
    kg = jnp.exp(d2 * (-0.125))
    t = _dot_t(ct[...], cf[...]) + 1.0
    kc = t * t * t
    inter = _dot_t(mt[...], mf[...])
    km = inter / (mrs_t[...] + mrs_r[...] - inter + 1e-8)
    return kg, kc, km


def _norms_kernel(gt, gf, ct, cf, mt, mf, gsq_t, gsq_r, mrs_t, mrs_r,
                  sg, sc, sm):
    i = pl.program_id(0)
    kg, kc, km = _cell_kernels(gt, gf, ct, cf, mt, mf, gsq_t, gsq_r,
                               mrs_t, mrs_r)

    @pl.when(i == 0)
    def _():
        sg[...] = jnp.zeros_like(sg)
        sc[...] = jnp.zeros_like(sc)
        sm[...] = jnp.zeros_like(sm)

    sg[...] += jnp.sum(kg * kg, keepdims=True)
    sc[...] += jnp.sum(kc * kc, keepdims=True)
    sm[...] += jnp.sum(km * km, keepdims=True)


def _topk_filter(fused, w):
    mask = jnp.zeros(w.shape, jnp.bool_)
    colidx = jax.lax.broadcasted_iota(jnp.int32, w.shape, 1)
    ninf = jnp.float32(-jnp.inf)
    for _ in range(TOPK):
        mx = jnp.max(w, axis=1, keepdims=True)
        ismax = w >= mx
        first = jnp.min(jnp.where(ismax, colidx, w.shape[1]), axis=1,
                        keepdims=True)
        sel = colidx == first
        mask = jnp.logical_or(mask, sel)
        w = jnp.where(sel, ninf, w)
    return jnp.where(mask, fused, 0.0)


def _fuse_topk_kernel(gt, gf, ct, cf, mt, mf, gsq_t, gsq_r, mrs_t, mrs_r,
                      cvec, att_r, filt, rsum):
    kg, kc, km = _cell_kernels(gt, gf, ct, cf, mt, mf, gsq_t, gsq_r,
                               mrs_t, mrs_r)
    cv = cvec[...]
    fused = cv[:, 0:1] * kg + cv[:, 1:2] * kc + cv[:, 2:3] * km
    f = _topk_filter(fused, fused * att_r[...])
    filt[...] = f
    rsum[...] = jnp.sum(f, axis=1, keepdims=True)


def _drug_topk_kernel(ft, ff, drs_t, drs_r, att_r, filt, rsum):
    inter = _dot_t(ft[...], ff[...])
    sim = inter / (drs_t[...] + drs_r[...] - inter + 1e-8)
    f = _topk_filter(sim, sim * att_r[...])
    filt[...] = f
    rsum[...] = jnp.sum(f, axis=1, keepdims=True)


def _gcn_kernel(fc, fdm, cd, cdT, rsc, rsd,
                k2cW, k2cb, k2dW, k2db, eW1, eW2, dW, pred):
    fcv = fc[...]
    fdv = fdm[...]
    cdv = cd[...]
    cdTv = cdT[...]
    dc = jax.lax.rsqrt(rsc[...] + jnp.sum(cdv, axis=1, keepdims=True) + 1e-9)
    dd = jax.lax.rsqrt(rsd[...] + jnp.sum(cdTv, axis=1, keepdims=True) + 1e-9)
    cfeat = _dot(fcv, k2cW[...]) + k2cb[...]
    dfeat = _dot(fdv, k2dW[...]) + k2db[...]
    eW1v = eW1[...]
    Zc = _dot(cfeat, eW1v)
    Zd = _dot(dfeat, eW1v)
    Yc = dc * Zc
    Yd = dd * Zd
    hc = jnp.maximum(Zc + dc * (_dot(fcv, Yc) + _dot(cdv, Yd)), 0.0)
    hd = jnp.maximum(Zd + dd * (_dot(cdTv, Yc) + _dot(fdv, Yd)), 0.0)
    eW2v = eW2[...]
    Zc2 = _dot(hc, eW2v)
    Zd2 = _dot(hd, eW2v)
    Yc2 = dc * Zc2
    Yd2 = dd * Zd2
    ec = Zc2 + dc * (_dot(fcv, Yc2) + _dot(cdv, Yd2))
    ed = Zd2 + dd * (_dot(cdTv, Yc2) + _dot(fdv, Yd2))
    pred[...] = jax.nn.sigmoid(_dot_t(_dot(ec, dW[...]), ed))


def kernel(gene, cna, mutation, feature_drug, cell_drug_adj,
           Wg, bg, Wc, bc, Wm, bm, Wd, bd,
           att_W1, att_b1, att_W2, att_b2, static_w,
           cW1, cb1, cW2, cb2, dW1, db1, dW2, db2,
           k2c_W, k2c_b, k2d_W, k2d_b, enc_W1, enc_W2, dec_W):
    r2 = lambda v: v.reshape(1, -1)
    sds = jax.ShapeDtypeStruct

    gene_n, cna_n, coef, catt, datt, gsq, mrs, drs = pl.pallas_call(
        _prep_kernel,
        out_shape=[
            sds((N_CELL, gene.shape[1]), F32),
            sds((N_CELL, cna.shape[1]), F32),
            sds((N_CELL, 3), F32),
            sds((N_CELL, 1), F32),
            sds((N_DRUG, 1), F32),
            sds((N_CELL, 1), F32),
            sds((N_CELL, 1), F32),
            sds((N_DRUG, 1), F32),
        ],
    )(gene, cna, mutation, feature_drug,
      Wg, r2(bg), Wc, r2(bc), Wm, r2(bm), Wd, r2(bd),
      att_W1, r2(att_b1), att_W2, r2(att_b2), r2(static_w),
      cW1, r2(cb1), cW2, r2(cb2), dW1, r2(db1), dW2, r2(db2))

    gsq_r = gsq.reshape(1, -1)
    mrs_r = mrs.reshape(1, -1)

    def tile_spec(dim):
        return pl.BlockSpec((TILE, dim), lambda i: (i, 0))

    def full_spec(rows, dim):
        return pl.BlockSpec((rows, dim), lambda i: (0, 0))

    GD = gene.shape[1]
    CD = cna.shape[1]
    MD = mutation.shape[1]
    ntiles = N_CELL // TILE

    cell_pass_specs = [
        tile_spec(GD), full_spec(N_CELL, GD),
        tile_spec(CD), full_spec(N_CELL, CD),
        tile_spec(MD), full_spec(N_CELL, MD),
        tile_spec(1), full_spec(1, N_CELL),
        tile_spec(1), full_spec(1, N_CELL),
    ]
    cell_pass_args = (gene_n, gene_n, cna_n, cna_n, mutation, mutation,
                      gsq, gsq_r, mrs, mrs_r)

    sg, sc, sm = pl.pallas_call(
        _norms_kernel,
        grid=(ntiles,),
        in_specs=cell_pass_specs,
        out_specs=[pl.BlockSpec((1, 1), lambda i: (0, 0))] * 3,
        out_shape=[sds((1, 1), F32)] * 3,
    )(*cell_pass_args)

    norms = jnp.sqrt(jnp.concatenate([sg, sc, sm], axis=1))
    cvec = coef / (norms + 1e-8)

    filt_c, rsum_c = pl.pallas_call(
        _fuse_topk_kernel,
        grid=(ntiles,),
        in_specs=cell_pass_specs + [tile_spec(3), full_spec(1, N_CELL)],
        out_specs=[tile_spec(N_CELL), tile_spec(1)],
        out_shape=[sds((N_CELL, N_CELL), F32), sds((N_CELL, 1), F32)],
    )(*cell_pass_args, cvec, catt.reshape(1, -1))

    DD = feature_drug.shape[1]
    drs_r = drs.reshape(1, -1)
    filt_d, rsum_d = pl.pallas_call(
        _drug_topk_kernel,
        grid=(N_DRUG // TILE,),
        in_specs=[
            tile_spec(DD), full_spec(N_DRUG, DD),
            tile_spec(1), full_spec(1, N_DRUG),
            full_spec(1, N_DRUG),
        ],
        out_specs=[tile_spec(N_DRUG), tile_spec(1)],
        out_shape=[sds((N_DRUG, N_DRUG), F32), sds((N_DRUG, 1), F32)],
    )(feature_drug, feature_drug, drs, drs_r, datt.reshape(1, -1))

    pred = pl.pallas_call(
        _gcn_kernel,
        out_shape=sds((N_CELL, N_DRUG), F32),
        compiler_params=pltpu.CompilerParams(
            vmem_limit_bytes=128 * 1024 * 1024),
    )(filt_c, filt_d, cell_drug_adj, cell_drug_adj.T, rsum_c, rsum_d,
      k2c_W, r2(k2c_b), k2d_W, r2(k2d_b), enc_W1, enc_W2, dec_W)
    return pred


# speedup vs baseline: 8.6551x; 1.1202x over previous
import jax
import jax.numpy as jnp
from jax.experimental import pallas as pl
from jax.experimental.pallas import tpu as pltpu

N_CELL = 2048
N_DRUG = 1024
TILE = 256
TOPK = 10
F32 = jnp.float32


def _dot(a, b):
    return jax.lax.dot_general(a, b, (((1,), (0,)), ((), ())),
                               preferred_element_type=F32)


def _dot_t(a, b):
    return jax.lax.dot_general(a, b, (((1,), (1,)), ((), ())),
                               preferred_element_type=F32)


def _prep_kernel(gene, cna, mut, fd,
                 Wg, bg, Wc, bc, Wm, bm, Wd, bd,
                 aW1, ab1, aW2, ab2, sw,
                 cW1, cb1, cW2, cb2, dW1, db1, dW2, db2,
                 gene_n, cna_n, coef, catt, datt, gsq, mrs, drs):
    g = gene[...]
    gm = jnp.mean(g, axis=0, keepdims=True)
    gv = jnp.mean((g - gm) * (g - gm), axis=0, keepdims=True)
    gn = (g - gm) / (jnp.sqrt(gv) + 1e-8)
    gene_n[...] = gn
    c = cna[...]
    cm = jnp.mean(c, axis=0, keepdims=True)
    cv = jnp.mean((c - cm) * (c - cm), axis=0, keepdims=True)
    cn = (c - cm) / (jnp.sqrt(cv) + 1e-8)
    cna_n[...] = cn
    m = mut[...]
    ge = _dot(gn, Wg[...]) + bg[...]
    ce = _dot(cn, Wc[...]) + bc[...]
    me = _dot(m, Wm[...]) + bm[...]
    shared = (ge + ce + me) / 3.0
    logits = _dot(jnp.maximum(_dot(shared, aW1[...]) + ab1[...], 0.0),
                  aW2[...]) + ab2[...]
    z = logits - jnp.max(logits, axis=1, keepdims=True)
    e = jnp.exp(z)
    dyn = e / jnp.sum(e, axis=1, keepdims=True)
    swv = sw[...]
    se = jnp.exp(swv - jnp.max(swv, axis=1, keepdims=True))
    st = se / jnp.sum(se, axis=1, keepdims=True)
    coef[...] = dyn * st
    s = jax.nn.sigmoid(_dot(jnp.maximum(_dot(shared, cW1[...]) + cb1[...], 0.0),
                            cW2[...]) + cb2[...])
    catt[...] = s / (jnp.sum(s) + 1e-8)
    f = fd[...]
    de = _dot(f, Wd[...]) + bd[...]
    s2 = jax.nn.sigmoid(_dot(jnp.maximum(_dot(de, dW1[...]) + db1[...], 0.0),
                             dW2[...]) + db2[...])
    datt[...] = s2 / (jnp.sum(s2) + 1e-8)
    gsq[...] = jnp.sum(gn * gn, axis=1, keepdims=True)
    mrs[...] = jnp.sum(m, axis=1, keepdims=True)
    drs[...] = jnp.sum(f, axis=1, keepdims=True)


def _cell_kernels(gt, gf, ct, cf, mt, mf, gsq_t, gsq_r, mrs_t, mrs_r):
    d2 = jnp.maximum(gsq_t[...] + gsq_r[...] - 2.0 * _dot_t(gt[...], gf[...]),
                     0.0)
    kg = jnp.exp(d2 * (-0.125))
    t = _dot_t(ct[...], cf[...]) + 1.0
    kc = t * t * t
    inter = _dot_t(mt[...], mf[...])
    km = inter / (mrs_t[...] + mrs_r[...] - inter + 1e-8)
    return kg, kc, km


def _norms_kernel(gt, gf, ct, cf, mt, mf, gsq_t, gsq_r, mrs_t, mrs_r,
                  sg, sc, sm):
    i = pl.program_id(0)
    kg, kc, km = _cell_kernels(gt, gf, ct, cf, mt, mf, gsq_t, gsq_r,
                               mrs_t, mrs_r)

    @pl.when(i == 0)
    def _():
        sg[...] = jnp.zeros_like(sg)
        sc[...] = jnp.zeros_like(sc)
        sm[...] = jnp.zeros_like(sm)

    sg[...] += jnp.sum(kg * kg, keepdims=True)
    sc[...] += jnp.sum(kc * kc, keepdims=True)
    sm[...] += jnp.sum(km * km, keepdims=True)


def _topk_filter(fused, w):
    colidx = jax.lax.broadcasted_iota(jnp.int32, w.shape, 1)
    ninf = jnp.float32(-jnp.inf)
    for _ in range(TOPK):
        mx = jnp.max(w, axis=1, keepdims=True)
        ismax = w >= mx
        first = jnp.min(jnp.where(ismax, colidx, w.shape[1]), axis=1,
                        keepdims=True)
        w = jnp.where(colidx == first, ninf, w)
    return jnp.where(w == ninf, fused, 0.0)


def _fuse_topk_kernel(gt, gf, ct, cf, mt, mf, gsq_t, gsq_r, mrs_t, mrs_r,
                      cvec, att_r, filt, rsum):
    kg, kc, km = _cell_kernels(gt, gf, ct, cf, mt, mf, gsq_t, gsq_r,
                               mrs_t, mrs_r)
    cv = cvec[...]
    fused = cv[:, 0:1] * kg + cv[:, 1:2] * kc + cv[:, 2:3] * km
    f = _topk_filter(fused, fused * att_r[...])
    filt[...] = f
    rsum[...] = jnp.sum(f, axis=1, keepdims=True)


def _drug_topk_kernel(ft, ff, drs_t, drs_r, att_r, filt, rsum):
    inter = _dot_t(ft[...], ff[...])
    sim = inter / (drs_t[...] + drs_r[...] - inter + 1e-8)
    f = _topk_filter(sim, sim * att_r[...])
    filt[...] = f
    rsum[...] = jnp.sum(f, axis=1, keepdims=True)


def _gcn_kernel(fc, fdm, cd, cdT, rsc, rsd,
                k2cW, k2cb, k2dW, k2db, eW1, eW2, dW, pred):
    fcv = fc[...]
    fdv = fdm[...]
    cdv = cd[...]
    cdTv = cdT[...]
    dc = jax.lax.rsqrt(rsc[...] + jnp.sum(cdv, axis=1, keepdims=True) + 1e-9)
    dd = jax.lax.rsqrt(rsd[...] + jnp.sum(cdTv, axis=1, keepdims=True) + 1e-9)
    cfeat = _dot(fcv, k2cW[...]) + k2cb[...]
    dfeat = _dot(fdv, k2dW[...]) + k2db[...]
    eW1v = eW1[...]
    Zc = _dot(cfeat, eW1v)
    Zd = _dot(dfeat, eW1v)
    Yc = dc * Zc
    Yd = dd * Zd
    hc = jnp.maximum(Zc + dc * (_dot(fcv, Yc) + _dot(cdv, Yd)), 0.0)
    hd = jnp.maximum(Zd + dd * (_dot(cdTv, Yc) + _dot(fdv, Yd)), 0.0)
    eW2v = eW2[...]
    Zc2 = _dot(hc, eW2v)
    Zd2 = _dot(hd, eW2v)
    Yc2 = dc * Zc2
    Yd2 = dd * Zd2
    ec = Zc2 + dc * (_dot(fcv, Yc2) + _dot(cdv, Yd2))
    ed = Zd2 + dd * (_dot(cdTv, Yc2) + _dot(fdv, Yd2))
    pred[...] = jax.nn.sigmoid(_dot_t(_dot(ec, dW[...]), ed))


def kernel(gene, cna, mutation, feature_drug, cell_drug_adj,
           Wg, bg, Wc, bc, Wm, bm, Wd, bd,
           att_W1, att_b1, att_W2, att_b2, static_w,
           cW1, cb1, cW2, cb2, dW1, db1, dW2, db2,
           k2c_W, k2c_b, k2d_W, k2d_b, enc_W1, enc_W2, dec_W):
    r2 = lambda v: v.reshape(1, -1)
    sds = jax.ShapeDtypeStruct

    gene_n, cna_n, coef, catt, datt, gsq, mrs, drs = pl.pallas_call(
        _prep_kernel,
        out_shape=[
            sds((N_CELL, gene.shape[1]), F32),
            sds((N_CELL, cna.shape[1]), F32),
            sds((N_CELL, 3), F32),
            sds((N_CELL, 1), F32),
            sds((N_DRUG, 1), F32),
            sds((N_CELL, 1), F32),
            sds((N_CELL, 1), F32),
            sds((N_DRUG, 1), F32),
        ],
    )(gene, cna, mutation, feature_drug,
      Wg, r2(bg), Wc, r2(bc), Wm, r2(bm), Wd, r2(bd),
      att_W1, r2(att_b1), att_W2, r2(att_b2), r2(static_w),
      cW1, r2(cb1), cW2, r2(cb2), dW1, r2(db1), dW2, r2(db2))

    gsq_r = gsq.reshape(1, -1)
    mrs_r = mrs.reshape(1, -1)

    def tile_spec(dim):
        return pl.BlockSpec((TILE, dim), lambda i: (i, 0))

    def full_spec(rows, dim):
        return pl.BlockSpec((rows, dim), lambda i: (0, 0))

    GD = gene.shape[1]
    CD = cna.shape[1]
    MD = mutation.shape[1]
    ntiles = N_CELL // TILE

    cell_pass_specs = [
        tile_spec(GD), full_spec(N_CELL, GD),
        tile_spec(CD), full_spec(N_CELL, CD),
        tile_spec(MD), full_spec(N_CELL, MD),
        tile_spec(1), full_spec(1, N_CELL),
        tile_spec(1), full_spec(1, N_CELL),
    ]
    cell_pass_args = (gene_n, gene_n, cna_n, cna_n, mutation, mutation,
                      gsq, gsq_r, mrs, mrs_r)

    sg, sc, sm = pl.pallas_call(
        _norms_kernel,
        grid=(ntiles,),
        in_specs=cell_pass_specs,
        out_specs=[pl.BlockSpec((1, 1), lambda i: (0, 0))] * 3,
        out_shape=[sds((1, 1), F32)] * 3,
    )(*cell_pass_args)

    norms = jnp.sqrt(jnp.concatenate([sg, sc, sm], axis=1))
    cvec = coef / (norms + 1e-8)

    filt_c, rsum_c = pl.pallas_call(
        _fuse_topk_kernel,
        grid=(ntiles,),
        in_specs=cell_pass_specs + [tile_spec(3), full_spec(1, N_CELL)],
        out_specs=[tile_spec(N_CELL), tile_spec(1)],
        out_shape=[sds((N_CELL, N_CELL), F32), sds((N_CELL, 1), F32)],
    )(*cell_pass_args, cvec, catt.reshape(1, -1))

    DD = feature_drug.shape[1]
    drs_r = drs.reshape(1, -1)
    filt_d, rsum_d = pl.pallas_call(
        _drug_topk_kernel,
        grid=(N_DRUG // TILE,),
        in_specs=[
            tile_spec(DD), full_spec(N_DRUG, DD),
            tile_spec(1), full_spec(1, N_DRUG),
            full_spec(1, N_DRUG),
        ],
        out_specs=[tile_spec(N_DRUG), tile_spec(1)],
        out_shape=[sds((N_DRUG, N_DRUG), F32), sds((N_DRUG, 1), F32)],
    )(feature_drug, feature_drug, drs, drs_r, datt.reshape(1, -1))

    pred = pl.pallas_call(
        _gcn_kernel,
        out_shape=sds((N_CELL, N_DRUG), F32),
        compiler_params=pltpu.CompilerParams(
            vmem_limit_bytes=128 * 1024 * 1024),
    )(filt_c, filt_d, cell_drug_adj, cell_drug_adj.T, rsum_c, rsum_d,
      k2c_W, r2(k2c_b), k2d_W, r2(k2d_b), enc_W1, enc_W2, dec_W)
    return pred


# speedup vs baseline: 10.1770x; 1.1758x over previous
import jax
import jax.numpy as jnp
from jax.experimental import pallas as pl
from jax.experimental.pallas import tpu as pltpu

N_CELL = 2048
N_DRUG = 1024
TILE = 256
TOPK = 10
F32 = jnp.float32
BF16 = jnp.bfloat16


def _dot(a, b):
    return jax.lax.dot_general(a, b, (((1,), (0,)), ((), ())),
                               preferred_element_type=F32)


def _dot_t(a, b):
    return jax.lax.dot_general(a, b, (((1,), (1,)), ((), ())),
                               preferred_element_type=F32)


def _prep_kernel(gene, cna, mut, fd,
                 Wg, bg, Wc, bc, Wm, bm, Wd, bd,
                 aW1, ab1, aW2, ab2, sw,
                 cW1, cb1, cW2, cb2, dW1, db1, dW2, db2,
                 gene_nb, cna_nb, coef, catt, datt, gsq, mrs, drs):
    g = gene[...]
    gm = jnp.mean(g, axis=0, keepdims=True)
    gv = jnp.mean((g - gm) * (g - gm), axis=0, keepdims=True)
    gn = (g - gm) / (jnp.sqrt(gv) + 1e-8)
    gene_nb[...] = gn.astype(BF16)
    c = cna[...]
    cm = jnp.mean(c, axis=0, keepdims=True)
    cv = jnp.mean((c - cm) * (c - cm), axis=0, keepdims=True)
    cn = (c - cm) / (jnp.sqrt(cv) + 1e-8)
    cna_nb[...] = cn.astype(BF16)
    m = mut[...]
    ge = _dot(gn, Wg[...]) + bg[...]
    ce = _dot(cn, Wc[...]) + bc[...]
    me = _dot(m, Wm[...]) + bm[...]
    shared = (ge + ce + me) / 3.0
    logits = _dot(jnp.maximum(_dot(shared, aW1[...]) + ab1[...], 0.0),
                  aW2[...]) + ab2[...]
    z = logits - jnp.max(logits, axis=1, keepdims=True)
    e = jnp.exp(z)
    dyn = e / jnp.sum(e, axis=1, keepdims=True)
    swv = sw[...]
    se = jnp.exp(swv - jnp.max(swv, axis=1, keepdims=True))
    st = se / jnp.sum(se, axis=1, keepdims=True)
    coef[...] = dyn * st
    s = jax.nn.sigmoid(_dot(jnp.maximum(_dot(shared, cW1[...]) + cb1[...], 0.0),
                            cW2[...]) + cb2[...])
    catt[...] = s / (jnp.sum(s) + 1e-8)
    f = fd[...]
    de = _dot(f, Wd[...]) + bd[...]
    s2 = jax.nn.sigmoid(_dot(jnp.maximum(_dot(de, dW1[...]) + db1[...], 0.0),
                             dW2[...]) + db2[...])
    datt[...] = s2 / (jnp.sum(s2) + 1e-8)
    gsq[...] = jnp.sum(gn * gn, axis=1, keepdims=True)
    mrs[...] = jnp.sum(m, axis=1, keepdims=True)
    drs[...] = jnp.sum(f, axis=1, keepdims=True)


def _cell_kernels(gt, gf, ct, cf, mt, mf, gsq_t, gsq_r, mrs_t, mrs_r):
    d2 = jnp.maximum(gsq_t[...] + gsq_r[...] - 2.0 * _dot_t(gt[...], gf[...]),
                     0.0)
    kg = jnp.exp(d2 * (-0.125))
    t = _dot_t(ct[...], cf[...]) + 1.0
    kc = t * t * t
    inter = _dot_t(mt[...], mf[...])
    km = inter / (mrs_t[...] + mrs_r[...] - inter + 1e-8)
    return kg, kc, km


def _norms_kernel(gt, gf, ct, cf, mt, mf, gsq_t, gsq_r, mrs_t, mrs_r,
                  sg, sc, sm):
    i = pl.program_id(0)
    kg, kc, km = _cell_kernels(gt, gf, ct, cf, mt, mf, gsq_t, gsq_r,
                               mrs_t, mrs_r)

    @pl.when(i == 0)
    def _():
        sg[...] = jnp.zeros_like(sg)
        sc[...] = jnp.zeros_like(sc)
        sm[...] = jnp.zeros_like(sm)

    sg[...] += jnp.sum(kg * kg, keepdims=True)
    sc[...] += jnp.sum(kc * kc, keepdims=True)
    sm[...] += jnp.sum(km * km, keepdims=True)


def _topk_filter(fused, w):
    b = jax.lax.bitcast_convert_type(w, jnp.int32)
    ikey = jnp.bitwise_xor(
        b, jnp.bitwise_and(jax.lax.shift_right_arithmetic(b, 31),
                           jnp.int32(0x7FFFFFFF)))
    colidx = jax.lax.broadcasted_iota(jnp.int32, w.shape, 1)
    key = jnp.bitwise_or(jnp.bitwise_and(ikey, jnp.int32(-2048)),
                         jnp.int32(2047) - colidx)
    sent = jnp.int32(-2147483648)
    for _ in range(TOPK):
        mx = jnp.max(key, axis=1, keepdims=True)
        key = jnp.where(key == mx, sent, key)
    return jnp.where(key == sent, fused, 0.0)


def _fuse_topk_kernel(gt, gf, ct, cf, mt, mf, gsq_t, gsq_r, mrs_t, mrs_r,
                      cvec, att_r, filt, rsum):
    kg, kc, km = _cell_kernels(gt, gf, ct, cf, mt, mf, gsq_t, gsq_r,
                               mrs_t, mrs_r)
    cv = cvec[...]
    fused = cv[:, 0:1] * kg + cv[:, 1:2] * kc + cv[:, 2:3] * km
    f = _topk_filter(fused, fused * att_r[...])
    filt[...] = f.astype(BF16)
    rsum[...] = jnp.sum(f, axis=1, keepdims=True)


def _drug_topk_kernel(ft, ff, drs_t, drs_r, att_r, filt, rsum):
    inter = _dot_t(ft[...], ff[...])
    sim = inter / (drs_t[...] + drs_r[...] - inter + 1e-8)
    f = _topk_filter(sim, sim * att_r[...])
    filt[...] = f.astype(BF16)
    rsum[...] = jnp.sum(f, axis=1, keepdims=True)


def _gcn_kernel(fc, fdm, cd, cdT, rsc, rsd,
                k2cW, k2cb, k2dW, k2db, eW1, eW2, dW, pred):
    fcv = fc[...]
    fdv = fdm[...]
    cdv = cd[...]
    cdTv = cdT[...]
    dc = jax.lax.rsqrt(rsc[...] + jnp.sum(cdv, axis=1, dtype=F32,
                                          keepdims=True) + 1e-9)
    dd = jax.lax.rsqrt(rsd[...] + jnp.sum(cdTv, axis=1, dtype=F32,
                                          keepdims=True) + 1e-9)
    cfeat = _dot(fcv, k2cW[...].astype(BF16)) + k2cb[...]
    dfeat = _dot(fdv, k2dW[...].astype(BF16)) + k2db[...]
    eW1v = eW1[...]
    Zc = _dot(cfeat, eW1v)
    Zd = _dot(dfeat, eW1v)
    Yc = (dc * Zc).astype(BF16)
    Yd = (dd * Zd).astype(BF16)
    hc = jnp.maximum(Zc + dc * (_dot(fcv, Yc) + _dot(cdv, Yd)), 0.0)
    hd = jnp.maximum(Zd + dd * (_dot(cdTv, Yc) + _dot(fdv, Yd)), 0.0)
    eW2v = eW2[...]
    Zc2 = _dot(hc, eW2v)
    Zd2 = _dot(hd, eW2v)
    Yc2 = (dc * Zc2).astype(BF16)
    Yd2 = (dd * Zd2).astype(BF16)
    ec = Zc2 + dc * (_dot(fcv, Yc2) + _dot(cdv, Yd2))
    ed = Zd2 + dd * (_dot(cdTv, Yc2) + _dot(fdv, Yd2))
    pred[...] = jax.nn.sigmoid(_dot_t(_dot(ec, dW[...]), ed))


def kernel(gene, cna, mutation, feature_drug, cell_drug_adj,
           Wg, bg, Wc, bc, Wm, bm, Wd, bd,
           att_W1, att_b1, att_W2, att_b2, static_w,
           cW1, cb1, cW2, cb2, dW1, db1, dW2, db2,
           k2c_W, k2c_b, k2d_W, k2d_b, enc_W1, enc_W2, dec_W):
    r2 = lambda v: v.reshape(1, -1)
    sds = jax.ShapeDtypeStruct

    gene_nb, cna_nb, coef, catt, datt, gsq, mrs, drs = pl.pallas_call(
        _prep_kernel,
        out_shape=[
            sds((N_CELL, gene.shape[1]), BF16),
            sds((N_CELL, cna.shape[1]), BF16),
            sds((N_CELL, 3), F32),
            sds((N_CELL, 1), F32),
            sds((N_DRUG, 1), F32),
            sds((N_CELL, 1), F32),
            sds((N_CELL, 1), F32),
            sds((N_DRUG, 1), F32),
        ],
    )(gene, cna, mutation, feature_drug,
      Wg, r2(bg), Wc, r2(bc), Wm, r2(bm), Wd, r2(bd),
      att_W1, r2(att_b1), att_W2, r2(att_b2), r2(static_w),
      cW1, r2(cb1), cW2, r2(cb2), dW1, r2(db1), dW2, r2(db2))

    mut_b = mutation.astype(BF16)
    gsq_r = gsq.reshape(1, -1)
    mrs_r = mrs.reshape(1, -1)

    def tile_spec(dim):
        return pl.BlockSpec((TILE, dim), lambda i: (i, 0))

    def full_spec(rows, dim):
        return pl.BlockSpec((rows, dim), lambda i: (0, 0))

    GD = gene.shape[1]
    CD = cna.shape[1]
    MD = mutation.shape[1]
    ntiles = N_CELL // TILE

    cell_pass_specs = [
        tile_spec(GD), full_spec(N_CELL, GD),
        tile_spec(CD), full_spec(N_CELL, CD),
        tile_spec(MD), full_spec(N_CELL, MD),
        tile_spec(1), full_spec(1, N_CELL),
        tile_spec(1), full_spec(1, N_CELL),
    ]
    cell_pass_args = (gene_nb, gene_nb, cna_nb, cna_nb, mut_b, mut_b,
                      gsq, gsq_r, mrs, mrs_r)

    sg, sc, sm = pl.pallas_call(
        _norms_kernel,
        grid=(ntiles,),
        in_specs=cell_pass_specs,
        out_specs=[pl.BlockSpec((1, 1), lambda i: (0, 0))] * 3,
        out_shape=[sds((1, 1), F32)] * 3,
    )(*cell_pass_args)

    norms = jnp.sqrt(jnp.concatenate([sg, sc, sm], axis=1))
    cvec = coef / (norms + 1e-8)

    filt_c, rsum_c = pl.pallas_call(
        _fuse_topk_kernel,
        grid=(ntiles,),
        in_specs=cell_pass_specs + [tile_spec(3), full_spec(1, N_CELL)],
        out_specs=[tile_spec(N_CELL), tile_spec(1)],
        out_shape=[sds((N_CELL, N_CELL), BF16), sds((N_CELL, 1), F32)],
    )(*cell_pass_args, cvec, catt.reshape(1, -1))

    DD = feature_drug.shape[1]
    fd_b = feature_drug.astype(BF16)
    drs_r = drs.reshape(1, -1)
    filt_d, rsum_d = pl.pallas_call(
        _drug_topk_kernel,
        grid=(N_DRUG // TILE,),
        in_specs=[
            tile_spec(DD), full_spec(N_DRUG, DD),
            tile_spec(1), full_spec(1, N_DRUG),
            full_spec(1, N_DRUG),
        ],
        out_specs=[tile_spec(N_DRUG), tile_spec(1)],
        out_shape=[sds((N_DRUG, N_DRUG), BF16), sds((N_DRUG, 1), F32)],
    )(fd_b, fd_b, drs, drs_r, datt.reshape(1, -1))

    cd_b = cell_drug_adj.astype(BF16)
    pred = pl.pallas_call(
        _gcn_kernel,
        out_shape=sds((N_CELL, N_DRUG), F32),
        compiler_params=pltpu.CompilerParams(
            vmem_limit_bytes=128 * 1024 * 1024),
    )(filt_c, filt_d, cd_b, cd_b.T, rsum_c, rsum_d,
      k2c_W, r2(k2c_b), k2d_W, r2(k2d_b), enc_W1, enc_W2, dec_W)
    return pred


# speedup vs baseline: 10.8987x; 1.0709x over previous
import jax
import jax.numpy as jnp
from jax.experimental import pallas as pl
from jax.experimental.pallas import tpu as pltpu

N_CELL = 2048
N_DRUG = 1024
TILE = 256
TOPK = 10
F32 = jnp.float32
BF16 = jnp.bfloat16


def _dot(a, b):
    return jax.lax.dot_general(a, b, (((1,), (0,)), ((), ())),
                               preferred_element_type=F32)


def _dot_t(a, b):
    return jax.lax.dot_general(a, b, (((1,), (1,)), ((), ())),
                               preferred_element_type=F32)


def _dot_tn(a, b):
    return jax.lax.dot_general(a, b, (((0,), (0,)), ((), ())),
                               preferred_element_type=F32)


def _prep_kernel(gene, cna, mut, fd,
                 Wg, bg, Wc, bc, Wm, bm, Wd, bd,
                 aW1, ab1, aW2, ab2, sw,
                 cW1, cb1, cW2, cb2, dW1, db1, dW2, db2,
                 cda,
                 gene_nb, cna_nb, coef, catt, datt, gsq, mrs, drs,
                 mut_b, fd_b, cd_b):
    g = gene[...]
    gm = jnp.mean(g, axis=0, keepdims=True)
    gv = jnp.mean((g - gm) * (g - gm), axis=0, keepdims=True)
    gn = (g - gm) / (jnp.sqrt(gv) + 1e-8)
    gene_nb[...] = gn.astype(BF16)
    c = cna[...]
    cm = jnp.mean(c, axis=0, keepdims=True)
    cv = jnp.mean((c - cm) * (c - cm), axis=0, keepdims=True)
    cn = (c - cm) / (jnp.sqrt(cv) + 1e-8)
    cna_nb[...] = cn.astype(BF16)
    m = mut[...]
    ge = _dot(gn, Wg[...]) + bg[...]
    ce = _dot(cn, Wc[...]) + bc[...]
    me = _dot(m, Wm[...]) + bm[...]
    shared = (ge + ce + me) / 3.0
    logits = _dot(jnp.maximum(_dot(shared, aW1[...]) + ab1[...], 0.0),
                  aW2[...]) + ab2[...]
    z = logits - jnp.max(logits, axis=1, keepdims=True)
    e = jnp.exp(z)
    dyn = e / jnp.sum(e, axis=1, keepdims=True)
    swv = sw[...]
    se = jnp.exp(swv - jnp.max(swv, axis=1, keepdims=True))
    st = se / jnp.sum(se, axis=1, keepdims=True)
    coef[...] = dyn * st
    s = jax.nn.sigmoid(_dot(jnp.maximum(_dot(shared, cW1[...]) + cb1[...], 0.0),
                            cW2[...]) + cb2[...])
    catt[...] = s / (jnp.sum(s) + 1e-8)
    f = fd[...]
    de = _dot(f, Wd[...]) + bd[...]
    s2 = jax.nn.sigmoid(_dot(jnp.maximum(_dot(de, dW1[...]) + db1[...], 0.0),
                             dW2[...]) + db2[...])
    datt[...] = s2 / (jnp.sum(s2) + 1e-8)
    gsq[...] = jnp.sum(gn * gn, axis=1, keepdims=True)
    mrs[...] = jnp.sum(m, axis=1, keepdims=True)
    drs[...] = jnp.sum(f, axis=1, keepdims=True)
    mut_b[...] = m.astype(BF16)
    fd_b[...] = f.astype(BF16)
    cd_b[...] = cda[...].astype(BF16)


def _cell_dots(gt, gf, ct, cf, mt, mf, gsq_t, gsq_r, mrs_t, mrs_r):
    nd2 = jnp.minimum(2.0 * _dot_t(gt[...], gf[...]) - gsq_t[...]
                      - gsq_r[...], 0.0)
    t = _dot_t(ct[...], cf[...]) + 1.0
    inter = _dot_t(mt[...], mf[...])
    km = inter / (mrs_t[...] + mrs_r[...] - inter + 1e-8)
    return nd2, t, km


def _topk_filter(fused, w):
    b = jax.lax.bitcast_convert_type(w, jnp.int32)
    ikey = jnp.bitwise_xor(
        b, jnp.bitwise_and(jax.lax.shift_right_arithmetic(b, 31),
                           jnp.int32(0x7FFFFFFF)))
    colidx = jax.lax.broadcasted_iota(jnp.int32, w.shape, 1)
    key = jnp.bitwise_or(jnp.bitwise_and(ikey, jnp.int32(-2048)),
                         jnp.int32(2047) - colidx)
    sent = jnp.int32(-2147483648)
    for _ in range(TOPK):
        mx = jnp.max(key, axis=1, keepdims=True)
        key = jnp.where(key == mx, sent, key)
    return jnp.where(key == sent, fused, 0.0)


def _fuse_topk_kernel(gt, gf, ct, cf, mt, mf, gsq_t, gsq_r, mrs_t, mrs_r,
                      cvec, att_r, filt, rsum, sg, sc, sm):
    i = pl.program_id(0)
    nt = pl.num_programs(0) // 2
    nd2, t, km = _cell_dots(gt, gf, ct, cf, mt, mf, gsq_t, gsq_r,
                            mrs_t, mrs_r)

    @pl.when(i == 0)
    def _():
        sg[...] = jnp.zeros_like(sg)
        sc[...] = jnp.zeros_like(sc)
        sm[...] = jnp.zeros_like(sm)

    @pl.when(i < nt)
    def _():
        t2 = t * t
        sg[...] += jnp.sum(jnp.exp(nd2 * 0.25), keepdims=True)
        sc[...] += jnp.sum(t2 * t2 * t2, keepdims=True)
        sm[...] += jnp.sum(km * km, keepdims=True)

    @pl.when(i >= nt)
    def _():
        kg = jnp.exp(nd2 * 0.125)
        kc = t * t * t
        cv = cvec[...]
        c0 = cv[:, 0:1] / (jnp.sqrt(sg[...]) + 1e-8)
        c1 = cv[:, 1:2] / (jnp.sqrt(sc[...]) + 1e-8)
        c2 = cv[:, 2:3] / (jnp.sqrt(sm[...]) + 1e-8)
        fused = c0 * kg + c1 * kc + c2 * km
        f = _topk_filter(fused, fused * att_r[...])
        filt[...] = f.astype(BF16)
        rsum[...] = jnp.sum(f, axis=1, keepdims=True)


def _drug_topk_kernel(ft, ff, drs_t, drs_r, att_r, filt, rsum):
    inter = _dot_t(ft[...], ff[...])
    sim = inter / (drs_t[...] + drs_r[...] - inter + 1e-8)
    f = _topk_filter(sim, sim * att_r[...])
    filt[...] = f.astype(BF16)
    rsum[...] = jnp.sum(f, axis=1, keepdims=True)


def _gcn_kernel(fc, fdm, cd, rsc, rsd,
                k2cW, k2cb, k2dW, k2db, eW1, eW2, dW, pred):
    fcv = fc[...]
    fdv = fdm[...]
    cdv = cd[...]
    dc = jax.lax.rsqrt(rsc[...] + jnp.sum(cdv, axis=1, dtype=F32,
                                          keepdims=True) + 1e-9)
    cd_colsum = _dot_tn(cdv, jnp.ones((cdv.shape[0], 1), BF16))
    dd = jax.lax.rsqrt(rsd[...] + cd_colsum + 1e-9)
    cfeat = _dot(fcv, k2cW[...].astype(BF16)) + k2cb[...]
    dfeat = _dot(fdv, k2dW[...].astype(BF16)) + k2db[...]
    eW1v = eW1[...]
    Zc = _dot(cfeat, eW1v)
    Zd = _dot(dfeat, eW1v)
    Yc = (dc * Zc).astype(BF16)
    Yd = (dd * Zd).astype(BF16)
    hc = jnp.maximum(Zc + dc * (_dot(fcv, Yc) + _dot(cdv, Yd)), 0.0)
    hd = jnp.maximum(Zd + dd * (_dot_tn(cdv, Yc) + _dot(fdv, Yd)), 0.0)
    eW2v = eW2[...]
    Zc2 = _dot(hc, eW2v)
    Zd2 = _dot(hd, eW2v)
    Yc2 = (dc * Zc2).astype(BF16)
    Yd2 = (dd * Zd2).astype(BF16)
    ec = Zc2 + dc * (_dot(fcv, Yc2) + _dot(cdv, Yd2))
    ed = Zd2 + dd * (_dot_tn(cdv, Yc2) + _dot(fdv, Yd2))
    pq = _dot(ec.astype(BF16), dW[...].astype(BF16)).astype(BF16)
    pred[...] = jax.nn.sigmoid(_dot_t(pq, ed.astype(BF16)))


def kernel(gene, cna, mutation, feature_drug, cell_drug_adj,
           Wg, bg, Wc, bc, Wm, bm, Wd, bd,
           att_W1, att_b1, att_W2, att_b2, static_w,
           cW1, cb1, cW2, cb2, dW1, db1, dW2, db2,
           k2c_W, k2c_b, k2d_W, k2d_b, enc_W1, enc_W2, dec_W):
    r2 = lambda v: v.reshape(1, -1)
    sds = jax.ShapeDtypeStruct

    (gene_nb, cna_nb, coef, catt, datt, gsq, mrs, drs,
     mut_b, fd_b, cd_b) = pl.pallas_call(
        _prep_kernel,
        out_shape=[
            sds((N_CELL, gene.shape[1]), BF16),
            sds((N_CELL, cna.shape[1]), BF16),
            sds((N_CELL, 3), F32),
            sds((N_CELL, 1), F32),
            sds((N_DRUG, 1), F32),
            sds((N_CELL, 1), F32),
            sds((N_CELL, 1), F32),
            sds((N_DRUG, 1), F32),
            sds((N_CELL, mutation.shape[1]), BF16),
            sds((N_DRUG, feature_drug.shape[1]), BF16),
            sds((N_CELL, N_DRUG), BF16),
        ],
    )(gene, cna, mutation, feature_drug,
      Wg, r2(bg), Wc, r2(bc), Wm, r2(bm), Wd, r2(bd),
      att_W1, r2(att_b1), att_W2, r2(att_b2), r2(static_w),
      cW1, r2(cb1), cW2, r2(cb2), dW1, r2(db1), dW2, r2(db2),
      cell_drug_adj)

    gsq_r = gsq.reshape(1, -1)
    mrs_r = mrs.reshape(1, -1)

    ntiles = N_CELL // TILE

    def tile_spec(dim):
        return pl.BlockSpec((TILE, dim), lambda i: (i % ntiles, 0))

    def full_spec(rows, dim):
        return pl.BlockSpec((rows, dim), lambda i: (0, 0))

    def out_spec(dim):
        return pl.BlockSpec((TILE, dim),
                            lambda i: (jnp.maximum(i - ntiles, 0), 0))

    GD = gene.shape[1]
    CD = cna.shape[1]
    MD = mutation.shape[1]

    cell_pass_specs = [
        tile_spec(GD), full_spec(N_CELL, GD),
        tile_spec(CD), full_spec(N_CELL, CD),
        tile_spec(MD), full_spec(N_CELL, MD),
        tile_spec(1), full_spec(1, N_CELL),
        tile_spec(1), full_spec(1, N_CELL),
    ]
    cell_pass_args = (gene_nb, gene_nb, cna_nb, cna_nb, mut_b, mut_b,
                      gsq, gsq_r, mrs, mrs_r)

    filt_c, rsum_c = pl.pallas_call(
        _fuse_topk_kernel,
        grid=(2 * ntiles,),
        in_specs=cell_pass_specs + [tile_spec(3), full_spec(1, N_CELL)],
        out_specs=[out_spec(N_CELL), out_spec(1)],
        out_shape=[sds((N_CELL, N_CELL), BF16), sds((N_CELL, 1), F32)],
        scratch_shapes=[pltpu.VMEM((1, 1), F32)] * 3,
    )(*cell_pass_args, coef, catt.reshape(1, -1))

    DD = feature_drug.shape[1]
    drs_r = drs.reshape(1, -1)
    filt_d, rsum_d = pl.pallas_call(
        _drug_topk_kernel,
        grid=(N_DRUG // TILE,),
        in_specs=[
            tile_spec(DD), full_spec(N_DRUG, DD),
            tile_spec(1), full_spec(1, N_DRUG),
            full_spec(1, N_DRUG),
        ],
        out_specs=[tile_spec(N_DRUG), tile_spec(1)],
        out_shape=[sds((N_DRUG, N_DRUG), BF16), sds((N_DRUG, 1), F32)],
    )(fd_b, fd_b, drs, drs_r, datt.reshape(1, -1))

    pred = pl.pallas_call(
        _gcn_kernel,
        out_shape=sds((N_CELL, N_DRUG), F32),
        compiler_params=pltpu.CompilerParams(
            vmem_limit_bytes=128 * 1024 * 1024),
    )(filt_c, filt_d, cd_b, rsum_c, rsum_d,
      k2c_W, r2(k2c_b), k2d_W, r2(k2d_b), enc_W1, enc_W2, dec_W)
    return pred
